# baseline (speedup 1.0000x reference)
import functools

import jax
import jax.numpy as jnp
from jax import lax
from jax.experimental import pallas as pl
from jax.experimental.pallas import tpu as pltpu
from jax.experimental.pallas import tpu_sc as plsc

M, D, B = 100000, 64, 16384
LANES = 16
NW = 32
CHUNK = 8192
ROUNDS = D // NW


def _transpose_body(idx_ref, src_ref, idxT_ref, srcT_ref):
    idxT_ref[...] = idx_ref[...].T
    srcT_ref[...] = src_ref[...].T


def _scatter_body(idxT_hbm, srcT_hbm, delta_hbm, acc_v, idx_v, src_v):
    wid = lax.axis_index("s") * 2 + lax.axis_index("c")
    zeros16 = jnp.zeros((LANES,), jnp.float32)

    def zero_body(i, carry):
        acc_v[pl.ds(i * LANES, LANES)] = zeros16
        return carry

    def scat_body(k, carry):
        iv = idx_v[pl.ds(k * LANES, LANES)]
        sv = src_v[pl.ds(k * LANES, LANES)]
        plsc.addupdate_scatter(acc_v, [iv], sv)
        return carry

    for r in range(ROUNDS):
        j = r * NW + wid
        lax.fori_loop(0, M // LANES, zero_body, 0, unroll=25)
        for c in range(B // CHUNK):
            off = j * B + c * CHUNK
            pltpu.sync_copy(idxT_hbm.at[pl.ds(off, CHUNK)], idx_v)
            pltpu.sync_copy(srcT_hbm.at[pl.ds(off, CHUNK)], src_v)
            lax.fori_loop(0, CHUNK // LANES, scat_body, 0, unroll=8)
        pltpu.sync_copy(acc_v, delta_hbm.at[pl.ds(j * M, M)])


def _add_body(x_ref, dT_ref, o_ref):
    o_ref[...] = x_ref[...] + dT_ref[...].T


def kernel(x, index, src):
    index = index.astype(jnp.int32)

    TB = 4096
    idxT, srcT = pl.pallas_call(
        _transpose_body,
        grid=(B // TB,),
        in_specs=[
            pl.BlockSpec((TB, D), lambda i: (i, 0)),
            pl.BlockSpec((TB, D), lambda i: (i, 0)),
        ],
        out_specs=[
            pl.BlockSpec((D, TB), lambda i: (0, i)),
            pl.BlockSpec((D, TB), lambda i: (0, i)),
        ],
        out_shape=[
            jax.ShapeDtypeStruct((D, B), jnp.int32),
            jax.ShapeDtypeStruct((D, B), jnp.float32),
        ],
    )(index, src)

    sc_scatter = functools.partial(
        pl.kernel,
        mesh=plsc.VectorSubcoreMesh(core_axis_name="c", subcore_axis_name="s"),
        out_type=jax.ShapeDtypeStruct((D * M,), jnp.float32),
        scratch_types=[
            pltpu.VMEM((M,), jnp.float32),
            pltpu.VMEM((CHUNK,), jnp.int32),
            pltpu.VMEM((CHUNK,), jnp.float32),
        ],
        compiler_params=pltpu.CompilerParams(needs_layout_passes=False),
    )(_scatter_body)
    delta = sc_scatter(idxT.reshape(D * B), srcT.reshape(D * B))
    delta = delta.reshape(D, M)

    MB = 4096
    out = pl.pallas_call(
        _add_body,
        grid=(pl.cdiv(M, MB),),
        in_specs=[
            pl.BlockSpec((MB, D), lambda i: (i, 0)),
            pl.BlockSpec((D, MB), lambda i: (0, i)),
        ],
        out_specs=pl.BlockSpec((MB, D), lambda i: (i, 0)),
        out_shape=jax.ShapeDtypeStruct((M, D), jnp.float32),
    )(x, delta)
    return out


# speedup vs baseline: 2.1669x; 2.1669x over previous
import functools

import jax
import jax.numpy as jnp
from jax import lax
from jax.experimental import pallas as pl
from jax.experimental.pallas import tpu as pltpu
from jax.experimental.pallas import tpu_sc as plsc

M, D, B = 100000, 64, 16384
LANES = 16
NW = 32
CHUNK = 8192
ROUNDS = D // NW
MP = 100096


def _scatter_body(idxT_hbm, srcT_hbm, delta_hbm, acc_v, idx_v, src_v):
    wid = lax.axis_index("s") * 2 + lax.axis_index("c")
    zeros16 = jnp.zeros((LANES,), jnp.float32)

    def zero_body(i, carry):
        acc_v[pl.ds(i * LANES, LANES)] = zeros16
        return carry

    def scat_body(k, carry):
        iv = idx_v[pl.ds(k * LANES, LANES)]
        sv = src_v[pl.ds(k * LANES, LANES)]
        plsc.addupdate_scatter(acc_v, [iv], sv)
        return carry

    for r in range(ROUNDS):
        j = r * NW + wid
        lax.fori_loop(0, M // LANES, zero_body, 0, unroll=25)
        for c in range(B // CHUNK):
            off = j * B + c * CHUNK
            pltpu.sync_copy(idxT_hbm.at[pl.ds(off, CHUNK)], idx_v)
            pltpu.sync_copy(srcT_hbm.at[pl.ds(off, CHUNK)], src_v)
            lax.fori_loop(0, CHUNK // LANES, scat_body, 0, unroll=8)
        pltpu.sync_copy(acc_v, delta_hbm.at[pl.ds(j * MP, M)])


RG = 8


def _addt_body(xT_ref, delta_ref, oT_ref, acc_ref, sem):
    j = pl.program_id(0)
    copies = [
        pltpu.make_async_copy(
            delta_ref.at[pl.ds((j * RG + r) * MP, MP)], acc_ref.at[r], sem)
        for r in range(RG)
    ]
    for cp in copies:
        cp.start()
    for cp in copies:
        cp.wait()
    oT_ref[...] = xT_ref[...] + acc_ref[:, :M]


def kernel(x, index, src):
    index = index.astype(jnp.int32)
    idxT = jnp.transpose(index).reshape(D * B)
    srcT = jnp.transpose(src).reshape(D * B)

    sc_scatter = functools.partial(
        pl.kernel,
        mesh=plsc.VectorSubcoreMesh(core_axis_name="c", subcore_axis_name="s"),
        out_type=jax.ShapeDtypeStruct((D * MP,), jnp.float32),
        scratch_types=[
            pltpu.VMEM((M,), jnp.float32),
            pltpu.VMEM((CHUNK,), jnp.int32),
            pltpu.VMEM((CHUNK,), jnp.float32),
        ],
        compiler_params=pltpu.CompilerParams(needs_layout_passes=False),
    )(_scatter_body)
    delta = sc_scatter(idxT, srcT)

    outT = pl.pallas_call(
        _addt_body,
        grid=(D // RG,),
        in_specs=[
            pl.BlockSpec((RG, M), lambda j: (j, 0)),
            pl.BlockSpec(memory_space=pl.ANY),
        ],
        out_specs=pl.BlockSpec((RG, M), lambda j: (j, 0)),
        out_shape=jax.ShapeDtypeStruct((D, M), jnp.float32),
        scratch_shapes=[
            pltpu.VMEM((RG, MP), jnp.float32),
            pltpu.SemaphoreType.DMA,
        ],
    )(jnp.transpose(x), delta)
    return jnp.transpose(outT)


# speedup vs baseline: 2.3863x; 1.1012x over previous
import functools

import jax
import jax.numpy as jnp
from jax import lax
from jax.experimental import pallas as pl
from jax.experimental.pallas import tpu as pltpu
from jax.experimental.pallas import tpu_sc as plsc

M, D, B = 100000, 64, 16384
LANES = 16
NW = 32
CHUNK = 8192
NC = 32
MP = 100096


def _scatter_half(c0, idxT_hbm, srcT_hbm, delta_hbm, acc_v, idx_v, src_v):
    wid = lax.axis_index("s") * 2 + lax.axis_index("c")
    j = c0 + wid
    zeros16 = jnp.zeros((LANES,), jnp.float32)

    def zero_body(i, carry):
        acc_v[pl.ds(i * LANES, LANES)] = zeros16
        return carry

    def scat_body(k, carry):
        iv = idx_v[pl.ds(k * LANES, LANES)]
        sv = src_v[pl.ds(k * LANES, LANES)]
        plsc.addupdate_scatter(acc_v, [iv], sv)
        return carry

    lax.fori_loop(0, M // LANES, zero_body, 0, unroll=25)
    for c in range(B // CHUNK):
        off = j * B + c * CHUNK
        pltpu.sync_copy(idxT_hbm.at[pl.ds(off, CHUNK)], idx_v)
        pltpu.sync_copy(srcT_hbm.at[pl.ds(off, CHUNK)], src_v)
        lax.fori_loop(0, CHUNK // LANES, scat_body, 0, unroll=8)
    pltpu.sync_copy(acc_v, delta_hbm.at[pl.ds(wid * MP, M)])


RG = 8


def _addt_core(xT_ref, delta_ref, oT_ref, acc_ref, sem):
    j = pl.program_id(0)
    copies = [
        pltpu.make_async_copy(
            delta_ref.at[pl.ds((j * RG + r) * MP, MP)], acc_ref.at[r], sem)
        for r in range(RG)
    ]
    for cp in copies:
        cp.start()
    for cp in copies:
        cp.wait()
    oT_ref[...] = xT_ref[...] + acc_ref[:, :M]


def _addt_a(xT_ref, delta_ref, oT_ref, acc_ref, sem):
    _addt_core(xT_ref, delta_ref, oT_ref, acc_ref, sem)


def _addt_b(xT_ref, delta_ref, prev_ref, oT_ref, acc_ref, sem):
    del prev_ref
    _addt_core(xT_ref, delta_ref, oT_ref, acc_ref, sem)


def kernel(x, index, src):
    index = index.astype(jnp.int32)
    idxT = jnp.transpose(index).reshape(D * B)
    srcT = jnp.transpose(src).reshape(D * B)

    def sc_half(c0):
        return pl.kernel(
            functools.partial(_scatter_half, c0),
            mesh=plsc.VectorSubcoreMesh(
                core_axis_name="c", subcore_axis_name="s"),
            out_type=jax.ShapeDtypeStruct((NC * MP,), jnp.float32),
            scratch_types=[
                pltpu.VMEM((M,), jnp.float32),
                pltpu.VMEM((CHUNK,), jnp.int32),
                pltpu.VMEM((CHUNK,), jnp.float32),
            ],
            compiler_params=pltpu.CompilerParams(needs_layout_passes=False),
        )(idxT, srcT)

    delta_a = sc_half(0)
    delta_b = sc_half(NC)
    xT = jnp.transpose(x)

    add_scratch = [
        pltpu.VMEM((RG, MP), jnp.float32),
        pltpu.SemaphoreType.DMA,
    ]
    half_grid = (NC // RG,)
    out0 = pl.pallas_call(
        _addt_a,
        grid=half_grid,
        in_specs=[
            pl.BlockSpec((RG, M), lambda j: (j, 0)),
            pl.BlockSpec(memory_space=pl.ANY),
        ],
        out_specs=pl.BlockSpec((RG, M), lambda j: (j, 0)),
        out_shape=jax.ShapeDtypeStruct((D, M), jnp.float32),
        scratch_shapes=add_scratch,
    )(xT, delta_a)
    outT = pl.pallas_call(
        _addt_b,
        grid=half_grid,
        in_specs=[
            pl.BlockSpec((RG, M), lambda j: (j + NC // RG, 0)),
            pl.BlockSpec(memory_space=pl.ANY),
            pl.BlockSpec(memory_space=pl.ANY),
        ],
        out_specs=pl.BlockSpec((RG, M), lambda j: (j + NC // RG, 0)),
        out_shape=jax.ShapeDtypeStruct((D, M), jnp.float32),
        input_output_aliases={2: 0},
        scratch_shapes=add_scratch,
    )(xT, delta_b, out0)
    return jnp.transpose(outT)
